# baseline (speedup 1.0000x reference)
import functools

import jax
import jax.numpy as jnp
from jax.experimental import pallas as pl
from jax.experimental.pallas import tpu as pltpu

_RT = 32


def _tc_stream(k_ref, pred_ref, out_ref, *, C):
    i = pl.program_id(0)
    C_al = (C // 128) * 128
    lane = jax.lax.broadcasted_iota(jnp.int32, (1, 128), 1)

    pks = []
    for r in range(_RT):
        c = k_ref[i * _RT + r]
        c0 = (c // 128) * 128
        chunk = pred_ref[pl.ds(r, 1), pl.ds(c0, 128)]
        is_l = lane == (c - c0)
        pks.append(jnp.where(is_l, chunk, -jnp.inf).max(axis=1, keepdims=True))
        pred_ref[pl.ds(r, 1), pl.ds(c0, 128)] = jnp.where(is_l, -jnp.inf, chunk)

    main = pred_ref[:, :C_al]
    m = jnp.max(main, axis=1)
    tail = pred_ref[:, C_al:]
    tmask = jax.lax.broadcasted_iota(jnp.int32, tail.shape, 1) < (C - C_al)
    m = jnp.maximum(m, jnp.where(tmask, tail, -jnp.inf).max(axis=1))

    pk = jnp.concatenate(pks, axis=0)
    out_ref[...] = m[:, None] - pk


def kernel(prediction, k):
    B, C = prediction.shape
    k2 = k.astype(jnp.int32)
    C_pad = ((C + 127) // 128) * 128
    out = pl.pallas_call(
        functools.partial(_tc_stream, C=C),
        grid=(B // _RT,),
        in_specs=[
            pl.BlockSpec(memory_space=pltpu.SMEM),
            pl.BlockSpec((_RT, C_pad), lambda i: (i, 0)),
        ],
        out_specs=pl.BlockSpec((_RT, 1), lambda i: (i, 0)),
        out_shape=jax.ShapeDtypeStruct((B, 1), jnp.float32),
        compiler_params=pltpu.CompilerParams(
            dimension_semantics=("arbitrary",),
        ),
    )(k2, prediction)
    return out.reshape(B)
